# baseline (speedup 1.0000x reference)
import functools

import jax
import jax.numpy as jnp
from jax import lax
from jax.experimental import pallas as pl
from jax.experimental.pallas import tpu as pltpu
from jax.experimental.pallas import tpu_sc as plsc

N = 10000
E = 160000
EPT = E // 16
K = 64
CM = EPT // K
TAIL = EPT - CM * K
CSPLIT = 80
NACC = 10240
SLICE = NACC // 16
CO = 624
REM = N - 16 * CO

_mesh = plsc.VectorSubcoreMesh(core_axis_name="c", subcore_axis_name="s")


@functools.partial(
    pl.kernel,
    out_type=jax.ShapeDtypeStruct((NACC,), jnp.float32),
    mesh=_mesh,
    scratch_types=[
        pltpu.VMEM((K,), jnp.int32),
        pltpu.VMEM((K,), jnp.int32),
        pltpu.VMEM((K,), jnp.int32),
        pltpu.VMEM((TAIL,), jnp.int32),
        pltpu.VMEM((K,), jnp.float32),
        pltpu.VMEM((TAIL,), jnp.float32),
        pltpu.VMEM((SLICE,), jnp.float32),
        pltpu.VMEM_SHARED((NACC,), jnp.float32),
        pltpu.SemaphoreType.DMA,
        pltpu.SemaphoreType.DMA,
        pltpu.SemaphoreType.DMA,
        pltpu.SemaphoreType.DMA,
    ],
)
def _deg_kernel(dst_hbm_v, out_hbm, d0, d1, d2, dt, ones_buf, ones_t,
                res_buf, acc_sh, m0, m1, m2, mt):
    c = lax.axis_index("c")
    s = lax.axis_index("s")
    didx = (d0, d1, d2)
    sems = (m0, m1, m2)
    base_e = s * EPT
    for i in range(K // 16):
        ones_buf[pl.ds(i * 16, 16)] = jnp.full((16,), 1.0, jnp.float32)
    ones_t[pl.ds(0, 16)] = jnp.full((16,), 1.0, jnp.float32)
    for i in range(SLICE // 16):
        res_buf[pl.ds(i * 16, 16)] = jnp.zeros((16,), jnp.float32)
    pltpu.sync_copy(res_buf, acc_sh.at[pl.ds(s * SLICE, SLICE)])
    plsc.subcore_barrier()

    def dst_row(j):
        return dst_hbm_v.at[pl.ds(base_e + j * K, K)]

    def idx_issue(j, b):
        pltpu.async_copy(dst_row(j), didx[b], sems[b])

    def idx_wait(b):
        pltpu.make_async_copy(dst_row(0), didx[b], sems[b]).wait()

    idx_issue(0, 0)
    idx_issue(1, 1)

    def body(t, carry):
        j0 = t * 3
        for b in range(3):
            j = j0 + b

            @pl.when(j + 2 < CM)
            def _():
                idx_issue(j + 2, (b + 2) % 3)

            idx_wait(b)
            pltpu.sync_copy(ones_buf, acc_sh.at[didx[b]], add=True)
        return carry

    lax.fori_loop(0, CM // 3, body, 0)
    pltpu.async_copy(dst_hbm_v.at[pl.ds(base_e + CM * K, TAIL)], dt, mt).wait()
    pltpu.sync_copy(ones_t, acc_sh.at[dt], add=True)
    plsc.subcore_barrier()
    pltpu.sync_copy(acc_sh.at[pl.ds(s * SLICE, SLICE)], res_buf)
    for i in range(SLICE // 16):
        res_buf[pl.ds(i * 16, 16)] = res_buf[pl.ds(i * 16, 16)] + 1.0

    @pl.when(c == 0)
    def _():
        pltpu.sync_copy(res_buf, out_hbm.at[pl.ds(s * SLICE, SLICE)])


def _edge_loop(g_hbm, src_row, dst_row, fix_idx, acc_sh, sidx, didx, rows,
               msi, mdi, mr, n):
    NS = 4

    def idx_issue(j, b):
        pltpu.async_copy(src_row(j), sidx[b], msi[b])
        pltpu.async_copy(dst_row(j), didx[b], mdi[b])

    def idx_wait(b):
        pltpu.make_async_copy(src_row(0), sidx[b], msi[b]).wait()
        pltpu.make_async_copy(dst_row(0), didx[b], mdi[b]).wait()

    def gather_issue(b):
        pltpu.async_copy(g_hbm.at[sidx[b]], rows[b], mr[b])

    for j in range(3):
        idx_issue(j, j)
    for j in range(2):
        idx_wait(j)
        fix_idx(j)
        gather_issue(j)

    def body(t, carry):
        j0 = t * NS
        for b in range(NS):
            j = j0 + b
            b2 = (b + 2) % NS
            b3 = (b + 3) % NS

            @pl.when(j + 3 < n)
            def _():
                idx_issue(j + 3, b3)

            @pl.when(j + 2 < n)
            def _():
                idx_wait(b2)
                fix_idx(b2)
                gather_issue(b2)

            pltpu.make_async_copy(g_hbm.at[sidx[b]], rows[b], mr[b]).wait()
            pltpu.sync_copy(rows[b], acc_sh.at[didx[b]], add=True)
        return carry

    lax.fori_loop(0, n // NS, body, 0)


_SCAT_SCRATCH = ([pltpu.VMEM((K,), jnp.int32)] * 4
                 + [pltpu.VMEM((K,), jnp.int32)] * 4
                 + [pltpu.VMEM((K, 128), jnp.float32)] * 4
                 + [pltpu.VMEM((TAIL,), jnp.int32)]
                 + [pltpu.VMEM((TAIL,), jnp.int32)]
                 + [pltpu.VMEM((TAIL, 128), jnp.float32)]
                 + [pltpu.VMEM_SHARED((NACC, 128), jnp.float32)]
                 + [pltpu.SemaphoreType.DMA] * 13)


def _tail(src_hbm, dst_hbm, g_hbm, acc_sh, st, dt, rt, mt, s, base):
    off = s * EPT + CM * K
    pltpu.async_copy(src_hbm.at[pl.ds(off, TAIL)], st, mt).wait()
    st[pl.ds(0, 16)] = st[pl.ds(0, 16)] + base
    pltpu.async_copy(dst_hbm.at[pl.ds(off, TAIL)], dt, mt).wait()
    pltpu.async_copy(g_hbm.at[st], rt, mt).wait()
    pltpu.sync_copy(rt, acc_sh.at[dt], add=True)


def _copy_out(acc_sh, out_hbm, s, base):
    pltpu.sync_copy(acc_sh.at[pl.ds(s * CO, CO)],
                    out_hbm.at[pl.ds(base + s * CO, CO)])

    @pl.when(s == 0)
    def _():
        pltpu.sync_copy(acc_sh.at[pl.ds(16 * CO, REM)],
                        out_hbm.at[pl.ds(base + 16 * CO, REM)])


@functools.partial(
    pl.kernel,
    out_type=jax.ShapeDtypeStruct((2 * N, 128), jnp.float32),
    mesh=_mesh,
    scratch_types=_SCAT_SCRATCH,
)
def _scatter_fsplit(g_hbm, src_hbm, dst_hbm, out_hbm,
                    s0, s1, s2, s3, d0, d1, d2, d3, r0, r1, r2, r3,
                    st, dt, rt, acc_sh,
                    a0, a1, a2, a3, e0, e1, e2, e3,
                    f0, f1, f2, f3, mt):
    c = lax.axis_index("c")
    s = lax.axis_index("s")
    base = c * N
    base_e = s * EPT
    sidx = (s0, s1, s2, s3)
    pltpu.sync_copy(g_hbm.at[pl.ds(base + s * CO, CO)],
                    acc_sh.at[pl.ds(s * CO, CO)])

    @pl.when(s == 0)
    def _():
        pltpu.sync_copy(g_hbm.at[pl.ds(base + 16 * CO, REM)],
                        acc_sh.at[pl.ds(16 * CO, REM)])

    plsc.subcore_barrier()

    def fix_idx(b):
        sb = sidx[b]
        for q in range(K // 16):
            sb[pl.ds(q * 16, 16)] = sb[pl.ds(q * 16, 16)] + base

    _edge_loop(g_hbm,
               lambda j: src_hbm.at[pl.ds(base_e + j * K, K)],
               lambda j: dst_hbm.at[pl.ds(base_e + j * K, K)],
               fix_idx,
               acc_sh, sidx, (d0, d1, d2, d3),
               (r0, r1, r2, r3), (a0, a1, a2, a3),
               (e0, e1, e2, e3), (f0, f1, f2, f3), CM)
    _tail(src_hbm, dst_hbm, g_hbm, acc_sh, st, dt, rt, mt, s, base)
    plsc.subcore_barrier()
    _copy_out(acc_sh, out_hbm, s, base)


@functools.partial(
    pl.kernel,
    out_type=jax.ShapeDtypeStruct((2 * N, 128), jnp.float32),
    mesh=_mesh,
    scratch_types=_SCAT_SCRATCH,
)
def _scatter_esplit(g_hbm, src_hbm, dst_hbm, out_hbm,
                    s0, s1, s2, s3, d0, d1, d2, d3, r0, r1, r2, r3,
                    st, dt, rt, acc_sh,
                    a0, a1, a2, a3, e0, e1, e2, e3,
                    f0, f1, f2, f3, mt):
    c = lax.axis_index("c")
    s = lax.axis_index("s")
    base = c * N
    j_off = c * CSPLIT
    n = CSPLIT - 4 * c
    base_e = s * EPT
    for r in range(K):
        for q in range(8):
            r0[r, pl.ds(q * 16, 16)] = jnp.zeros((16,), jnp.float32)
    for t in range(SLICE // K):
        pltpu.sync_copy(r0, acc_sh.at[pl.ds(s * SLICE + t * K, K)])
    plsc.subcore_barrier()
    _edge_loop(g_hbm,
               lambda j: src_hbm.at[pl.ds(base_e + (j_off + j) * K, K)],
               lambda j: dst_hbm.at[pl.ds(base_e + (j_off + j) * K, K)],
               lambda b: None,
               acc_sh, (s0, s1, s2, s3), (d0, d1, d2, d3),
               (r0, r1, r2, r3), (a0, a1, a2, a3),
               (e0, e1, e2, e3), (f0, f1, f2, f3), n)

    @pl.when(c == 1)
    def _():
        _tail(src_hbm, dst_hbm, g_hbm, acc_sh, st, dt, rt, mt, s, 0)

    plsc.subcore_barrier()
    _copy_out(acc_sh, out_hbm, s, base)


_RB = 1000
_RB2 = 2000


def _tc_first(x, deg, w1):
    f_in, f_out = w1.shape
    dh = f_out // 2

    def body(x_ref, deg_ref, w_ref, g_ref):
        m = jnp.dot(x_ref[...], w_ref[...], preferred_element_type=jnp.float32)
        g_ref[...] = m * lax.rsqrt(deg_ref[...])

    return pl.pallas_call(
        body,
        grid=(N // _RB, 2),
        in_specs=[
            pl.BlockSpec((_RB, f_in), lambda r, c: (r, 0)),
            pl.BlockSpec((_RB, 1), lambda r, c: (r, 0)),
            pl.BlockSpec((f_in, dh), lambda r, c: (0, c)),
        ],
        out_specs=pl.BlockSpec((_RB, dh), lambda r, c: (c * (N // _RB) + r, 0)),
        out_shape=jax.ShapeDtypeStruct((2 * N, dh), jnp.float32),
    )(x, deg, w1)


def _tc_second(s1, deg, b, w):
    f_in, f_out = w.shape

    def body(sa_ref, sb_ref, deg_ref, b_ref, w_ref, g_ref):
        dv = lax.rsqrt(deg_ref[...])
        h = jnp.concatenate([sa_ref[...], sb_ref[...]], axis=1) * dv + b_ref[...]
        h = jnp.maximum(h, 0.0)
        g_ref[...] = jnp.dot(h, w_ref[...],
                             preferred_element_type=jnp.float32) * dv

    return pl.pallas_call(
        body,
        grid=(N // _RB2,),
        in_specs=[
            pl.BlockSpec((_RB2, 128), lambda r: (r, 0)),
            pl.BlockSpec((_RB2, 128), lambda r: (N // _RB2 + r, 0)),
            pl.BlockSpec((_RB2, 1), lambda r: (r, 0)),
            pl.BlockSpec((1, f_in), lambda r: (0, 0)),
            pl.BlockSpec((f_in, f_out), lambda r: (0, 0)),
        ],
        out_specs=pl.BlockSpec((_RB2, f_out), lambda r: (r, 0)),
        out_shape=jax.ShapeDtypeStruct((N, f_out), jnp.float32),
    )(s1, s1, deg, b, w)


def _tc_third(p2, g, deg, b):
    f = g.shape[1]

    def body(pa_ref, pb_ref, g_ref, deg_ref, b_ref, o_ref):
        dv = lax.rsqrt(deg_ref[...])
        s = pa_ref[...] + pb_ref[...] + g_ref[...]
        h = jnp.maximum(s * dv + b_ref[...], 0.0)
        o_ref[...] = h * dv

    return pl.pallas_call(
        body,
        grid=(N // _RB2,),
        in_specs=[
            pl.BlockSpec((_RB2, f), lambda r: (r, 0)),
            pl.BlockSpec((_RB2, f), lambda r: (N // _RB2 + r, 0)),
            pl.BlockSpec((_RB2, f), lambda r: (r, 0)),
            pl.BlockSpec((_RB2, 1), lambda r: (r, 0)),
            pl.BlockSpec((1, f), lambda r: (0, 0)),
        ],
        out_specs=pl.BlockSpec((_RB2, f), lambda r: (r, 0)),
        out_shape=jax.ShapeDtypeStruct((N, f), jnp.float32),
    )(p2, p2, g, deg, b)


def _tc_last(p3, g, deg, w, b):
    f_in, f_out = w.shape

    def body(pa_ref, pb_ref, g_ref, deg_ref, w_ref, b_ref, o_ref):
        s = pa_ref[...] + pb_ref[...] + g_ref[...]
        m = jnp.dot(s, w_ref[...], preferred_element_type=jnp.float32)
        o_ref[...] = m * lax.rsqrt(deg_ref[...]) + b_ref[...]

    return pl.pallas_call(
        body,
        grid=(N // _RB2,),
        in_specs=[
            pl.BlockSpec((_RB2, f_in), lambda r: (r, 0)),
            pl.BlockSpec((_RB2, f_in), lambda r: (N // _RB2 + r, 0)),
            pl.BlockSpec((_RB2, f_in), lambda r: (r, 0)),
            pl.BlockSpec((_RB2, 1), lambda r: (r, 0)),
            pl.BlockSpec((f_in, f_out), lambda r: (0, 0)),
            pl.BlockSpec((1, f_out), lambda r: (0, 0)),
        ],
        out_specs=pl.BlockSpec((_RB2, f_out), lambda r: (r, 0)),
        out_shape=jax.ShapeDtypeStruct((N, f_out), jnp.float32),
    )(p3, p3, g, deg, w, b)


def kernel(x, edge_index, W1, b1, W2, b2, W3, b3):
    srcv = edge_index[0].astype(jnp.int32)
    dstv = edge_index[1].astype(jnp.int32)
    deg = _deg_kernel(dstv).reshape(NACC, 1)
    g1 = _tc_first(x, deg, W1)
    s1 = _scatter_fsplit(g1, srcv, dstv)
    g2 = _tc_second(s1, deg, b1.reshape(1, -1), W2)
    p2 = _scatter_esplit(g2, srcv, dstv)
    g3 = _tc_third(p2, g2, deg, b2.reshape(1, -1))
    p3 = _scatter_esplit(g3, srcv, dstv)
    return _tc_last(p3, g3, deg, W3, b3.reshape(1, -1))


# speedup vs baseline: 1.0360x; 1.0360x over previous
import functools

import jax
import jax.numpy as jnp
from jax import lax
from jax.experimental import pallas as pl
from jax.experimental.pallas import tpu as pltpu
from jax.experimental.pallas import tpu_sc as plsc

N = 10000
E = 160000
EPT = E // 16
K = 64
CM = EPT // K
TAIL = EPT - CM * K
CSPLIT = 80
KD = 128
CMD = EPT // KD
NACC = 10240
SLICE = NACC // 16
CO = 624
REM = N - 16 * CO

_mesh = plsc.VectorSubcoreMesh(core_axis_name="c", subcore_axis_name="s")


@functools.partial(
    pl.kernel,
    out_type=jax.ShapeDtypeStruct((NACC,), jnp.float32),
    mesh=_mesh,
    scratch_types=[
        pltpu.VMEM((KD,), jnp.int32),
        pltpu.VMEM((KD,), jnp.int32),
        pltpu.VMEM((KD,), jnp.int32),
        pltpu.VMEM((TAIL,), jnp.int32),
        pltpu.VMEM((KD,), jnp.float32),
        pltpu.VMEM((TAIL,), jnp.float32),
        pltpu.VMEM((SLICE,), jnp.float32),
        pltpu.VMEM_SHARED((NACC,), jnp.float32),
        pltpu.SemaphoreType.DMA,
        pltpu.SemaphoreType.DMA,
        pltpu.SemaphoreType.DMA,
        pltpu.SemaphoreType.DMA,
    ],
)
def _deg_kernel(dst_hbm_v, out_hbm, d0, d1, d2, dt, ones_buf, ones_t,
                res_buf, acc_sh, m0, m1, m2, mt):
    c = lax.axis_index("c")
    s = lax.axis_index("s")
    didx = (d0, d1, d2)
    sems = (m0, m1, m2)
    base_e = s * EPT
    for i in range(KD // 16):
        ones_buf[pl.ds(i * 16, 16)] = jnp.full((16,), 1.0, jnp.float32)
    ones_t[pl.ds(0, 16)] = jnp.full((16,), 1.0, jnp.float32)
    for i in range(SLICE // 16):
        res_buf[pl.ds(i * 16, 16)] = jnp.zeros((16,), jnp.float32)
    pltpu.sync_copy(res_buf, acc_sh.at[pl.ds(s * SLICE, SLICE)])
    plsc.subcore_barrier()

    def dst_row(j):
        return dst_hbm_v.at[pl.ds(base_e + j * KD, KD)]

    def idx_issue(j, b):
        pltpu.async_copy(dst_row(j), didx[b], sems[b])

    def idx_wait(b):
        pltpu.make_async_copy(dst_row(0), didx[b], sems[b]).wait()

    idx_issue(0, 0)
    idx_issue(1, 1)

    def body(t, carry):
        j0 = t * 3
        for b in range(3):
            j = j0 + b

            @pl.when(j + 2 < CMD)
            def _():
                idx_issue(j + 2, (b + 2) % 3)

            idx_wait(b)
            pltpu.sync_copy(ones_buf, acc_sh.at[didx[b]], add=True)
        return carry

    lax.fori_loop(0, CMD // 3, body, 0)
    pltpu.async_copy(dst_hbm_v.at[pl.ds(base_e + CMD * KD, TAIL)], dt, mt).wait()
    pltpu.sync_copy(ones_t, acc_sh.at[dt], add=True)
    plsc.subcore_barrier()
    pltpu.sync_copy(acc_sh.at[pl.ds(s * SLICE, SLICE)], res_buf)
    for i in range(SLICE // 16):
        res_buf[pl.ds(i * 16, 16)] = res_buf[pl.ds(i * 16, 16)] + 1.0

    @pl.when(c == 0)
    def _():
        pltpu.sync_copy(res_buf, out_hbm.at[pl.ds(s * SLICE, SLICE)])


def _edge_loop(g_hbm, src_row, dst_row, fix_idx, acc_sh, sidx, didx, rows,
               msi, mdi, mr, n):
    NS = 4

    def idx_issue(j, b):
        pltpu.async_copy(src_row(j), sidx[b], msi[b])
        pltpu.async_copy(dst_row(j), didx[b], mdi[b])

    def idx_wait(b):
        pltpu.make_async_copy(src_row(0), sidx[b], msi[b]).wait()
        pltpu.make_async_copy(dst_row(0), didx[b], mdi[b]).wait()

    def gather_issue(b):
        pltpu.async_copy(g_hbm.at[sidx[b]], rows[b], mr[b])

    for j in range(3):
        idx_issue(j, j)
    for j in range(2):
        idx_wait(j)
        fix_idx(j)
        gather_issue(j)

    def body(t, carry):
        j0 = t * NS
        for b in range(NS):
            j = j0 + b
            b2 = (b + 2) % NS
            b3 = (b + 3) % NS

            @pl.when(j + 3 < n)
            def _():
                idx_issue(j + 3, b3)

            @pl.when(j + 2 < n)
            def _():
                idx_wait(b2)
                fix_idx(b2)
                gather_issue(b2)

            pltpu.make_async_copy(g_hbm.at[sidx[b]], rows[b], mr[b]).wait()
            pltpu.sync_copy(rows[b], acc_sh.at[didx[b]], add=True)
        return carry

    lax.fori_loop(0, n // NS, body, 0)


_SCAT_SCRATCH = ([pltpu.VMEM((K,), jnp.int32)] * 4
                 + [pltpu.VMEM((K,), jnp.int32)] * 4
                 + [pltpu.VMEM((K, 128), jnp.float32)] * 4
                 + [pltpu.VMEM((TAIL,), jnp.int32)]
                 + [pltpu.VMEM((TAIL,), jnp.int32)]
                 + [pltpu.VMEM((TAIL, 128), jnp.float32)]
                 + [pltpu.VMEM_SHARED((NACC, 128), jnp.float32)]
                 + [pltpu.SemaphoreType.DMA] * 13)


def _tail(src_hbm, dst_hbm, g_hbm, acc_sh, st, dt, rt, mt, s, base):
    off = s * EPT + CM * K
    pltpu.async_copy(src_hbm.at[pl.ds(off, TAIL)], st, mt).wait()
    st[pl.ds(0, 16)] = st[pl.ds(0, 16)] + base
    pltpu.async_copy(dst_hbm.at[pl.ds(off, TAIL)], dt, mt).wait()
    pltpu.async_copy(g_hbm.at[st], rt, mt).wait()
    pltpu.sync_copy(rt, acc_sh.at[dt], add=True)


def _copy_out(acc_sh, out_hbm, s, base):
    pltpu.sync_copy(acc_sh.at[pl.ds(s * CO, CO)],
                    out_hbm.at[pl.ds(base + s * CO, CO)])

    @pl.when(s == 0)
    def _():
        pltpu.sync_copy(acc_sh.at[pl.ds(16 * CO, REM)],
                        out_hbm.at[pl.ds(base + 16 * CO, REM)])


@functools.partial(
    pl.kernel,
    out_type=jax.ShapeDtypeStruct((2 * N, 128), jnp.float32),
    mesh=_mesh,
    scratch_types=_SCAT_SCRATCH,
)
def _scatter_fsplit(g_hbm, src_hbm, dst_hbm, out_hbm,
                    s0, s1, s2, s3, d0, d1, d2, d3, r0, r1, r2, r3,
                    st, dt, rt, acc_sh,
                    a0, a1, a2, a3, e0, e1, e2, e3,
                    f0, f1, f2, f3, mt):
    c = lax.axis_index("c")
    s = lax.axis_index("s")
    base = c * N
    base_e = s * EPT
    sidx = (s0, s1, s2, s3)
    pltpu.sync_copy(g_hbm.at[pl.ds(base + s * CO, CO)],
                    acc_sh.at[pl.ds(s * CO, CO)])

    @pl.when(s == 0)
    def _():
        pltpu.sync_copy(g_hbm.at[pl.ds(base + 16 * CO, REM)],
                        acc_sh.at[pl.ds(16 * CO, REM)])

    plsc.subcore_barrier()

    def fix_idx(b):
        sb = sidx[b]
        for q in range(K // 16):
            sb[pl.ds(q * 16, 16)] = sb[pl.ds(q * 16, 16)] + base

    _edge_loop(g_hbm,
               lambda j: src_hbm.at[pl.ds(base_e + j * K, K)],
               lambda j: dst_hbm.at[pl.ds(base_e + j * K, K)],
               fix_idx,
               acc_sh, sidx, (d0, d1, d2, d3),
               (r0, r1, r2, r3), (a0, a1, a2, a3),
               (e0, e1, e2, e3), (f0, f1, f2, f3), CM)
    _tail(src_hbm, dst_hbm, g_hbm, acc_sh, st, dt, rt, mt, s, base)
    plsc.subcore_barrier()
    _copy_out(acc_sh, out_hbm, s, base)


@functools.partial(
    pl.kernel,
    out_type=jax.ShapeDtypeStruct((2 * N, 128), jnp.float32),
    mesh=_mesh,
    scratch_types=_SCAT_SCRATCH,
)
def _scatter_esplit(g_hbm, src_hbm, dst_hbm, out_hbm,
                    s0, s1, s2, s3, d0, d1, d2, d3, r0, r1, r2, r3,
                    st, dt, rt, acc_sh,
                    a0, a1, a2, a3, e0, e1, e2, e3,
                    f0, f1, f2, f3, mt):
    c = lax.axis_index("c")
    s = lax.axis_index("s")
    base = c * N
    j_off = c * CSPLIT
    n = CSPLIT - 4 * c
    base_e = s * EPT
    for r in range(K):
        for q in range(8):
            r0[r, pl.ds(q * 16, 16)] = jnp.zeros((16,), jnp.float32)
    for t in range(SLICE // K):
        pltpu.sync_copy(r0, acc_sh.at[pl.ds(s * SLICE + t * K, K)])
    plsc.subcore_barrier()
    _edge_loop(g_hbm,
               lambda j: src_hbm.at[pl.ds(base_e + (j_off + j) * K, K)],
               lambda j: dst_hbm.at[pl.ds(base_e + (j_off + j) * K, K)],
               lambda b: None,
               acc_sh, (s0, s1, s2, s3), (d0, d1, d2, d3),
               (r0, r1, r2, r3), (a0, a1, a2, a3),
               (e0, e1, e2, e3), (f0, f1, f2, f3), n)

    @pl.when(c == 1)
    def _():
        _tail(src_hbm, dst_hbm, g_hbm, acc_sh, st, dt, rt, mt, s, 0)

    plsc.subcore_barrier()
    _copy_out(acc_sh, out_hbm, s, base)


_RB = 1000
_RB2 = 2000


def _tc_first(x, deg, w1):
    f_in, f_out = w1.shape
    dh = f_out // 2

    def body(x_ref, deg_ref, w_ref, g_ref):
        m = jnp.dot(x_ref[...].astype(jnp.bfloat16),
                    w_ref[...].astype(jnp.bfloat16),
                    preferred_element_type=jnp.float32)
        g_ref[...] = m * lax.rsqrt(deg_ref[...])

    return pl.pallas_call(
        body,
        grid=(N // _RB, 2),
        in_specs=[
            pl.BlockSpec((_RB, f_in), lambda r, c: (r, 0)),
            pl.BlockSpec((_RB, 1), lambda r, c: (r, 0)),
            pl.BlockSpec((f_in, dh), lambda r, c: (0, c)),
        ],
        out_specs=pl.BlockSpec((_RB, dh), lambda r, c: (c * (N // _RB) + r, 0)),
        out_shape=jax.ShapeDtypeStruct((2 * N, dh), jnp.float32),
    )(x, deg, w1)


def _tc_second(s1, deg, b, w):
    f_in, f_out = w.shape

    def body(sa_ref, sb_ref, deg_ref, b_ref, w_ref, g_ref):
        dv = lax.rsqrt(deg_ref[...])
        h = jnp.concatenate([sa_ref[...], sb_ref[...]], axis=1) * dv + b_ref[...]
        h = jnp.maximum(h, 0.0)
        g_ref[...] = jnp.dot(h, w_ref[...],
                             preferred_element_type=jnp.float32) * dv

    return pl.pallas_call(
        body,
        grid=(N // _RB2,),
        in_specs=[
            pl.BlockSpec((_RB2, 128), lambda r: (r, 0)),
            pl.BlockSpec((_RB2, 128), lambda r: (N // _RB2 + r, 0)),
            pl.BlockSpec((_RB2, 1), lambda r: (r, 0)),
            pl.BlockSpec((1, f_in), lambda r: (0, 0)),
            pl.BlockSpec((f_in, f_out), lambda r: (0, 0)),
        ],
        out_specs=pl.BlockSpec((_RB2, f_out), lambda r: (r, 0)),
        out_shape=jax.ShapeDtypeStruct((N, f_out), jnp.float32),
    )(s1, s1, deg, b, w)


def _tc_third(p2, g, deg, b):
    f = g.shape[1]

    def body(pa_ref, pb_ref, g_ref, deg_ref, b_ref, o_ref):
        dv = lax.rsqrt(deg_ref[...])
        s = pa_ref[...] + pb_ref[...] + g_ref[...]
        h = jnp.maximum(s * dv + b_ref[...], 0.0)
        o_ref[...] = h * dv

    return pl.pallas_call(
        body,
        grid=(N // _RB2,),
        in_specs=[
            pl.BlockSpec((_RB2, f), lambda r: (r, 0)),
            pl.BlockSpec((_RB2, f), lambda r: (N // _RB2 + r, 0)),
            pl.BlockSpec((_RB2, f), lambda r: (r, 0)),
            pl.BlockSpec((_RB2, 1), lambda r: (r, 0)),
            pl.BlockSpec((1, f), lambda r: (0, 0)),
        ],
        out_specs=pl.BlockSpec((_RB2, f), lambda r: (r, 0)),
        out_shape=jax.ShapeDtypeStruct((N, f), jnp.float32),
    )(p2, p2, g, deg, b)


def _tc_last(p3, g, deg, w, b):
    f_in, f_out = w.shape

    def body(pa_ref, pb_ref, g_ref, deg_ref, w_ref, b_ref, o_ref):
        s = pa_ref[...] + pb_ref[...] + g_ref[...]
        m = jnp.dot(s, w_ref[...], preferred_element_type=jnp.float32)
        o_ref[...] = m * lax.rsqrt(deg_ref[...]) + b_ref[...]

    return pl.pallas_call(
        body,
        grid=(N // _RB2,),
        in_specs=[
            pl.BlockSpec((_RB2, f_in), lambda r: (r, 0)),
            pl.BlockSpec((_RB2, f_in), lambda r: (N // _RB2 + r, 0)),
            pl.BlockSpec((_RB2, f_in), lambda r: (r, 0)),
            pl.BlockSpec((_RB2, 1), lambda r: (r, 0)),
            pl.BlockSpec((f_in, f_out), lambda r: (0, 0)),
            pl.BlockSpec((1, f_out), lambda r: (0, 0)),
        ],
        out_specs=pl.BlockSpec((_RB2, f_out), lambda r: (r, 0)),
        out_shape=jax.ShapeDtypeStruct((N, f_out), jnp.float32),
    )(p3, p3, g, deg, w, b)


def kernel(x, edge_index, W1, b1, W2, b2, W3, b3):
    srcv = edge_index[0].astype(jnp.int32)
    dstv = edge_index[1].astype(jnp.int32)
    deg = _deg_kernel(dstv).reshape(NACC, 1)
    g1 = _tc_first(x, deg, W1)
    s1 = _scatter_fsplit(g1, srcv, dstv)
    g2 = _tc_second(s1, deg, b1.reshape(1, -1), W2)
    p2 = _scatter_esplit(g2, srcv, dstv)
    g3 = _tc_third(p2, g2, deg, b2.reshape(1, -1))
    p3 = _scatter_esplit(g3, srcv, dstv)
    return _tc_last(p3, g3, deg, W3, b3.reshape(1, -1))


# speedup vs baseline: 1.0566x; 1.0199x over previous
import functools

import jax
import jax.numpy as jnp
from jax import lax
from jax.experimental import pallas as pl
from jax.experimental.pallas import tpu as pltpu
from jax.experimental.pallas import tpu_sc as plsc

N = 10000
E = 160000
EPT = E // 16
K = 64
CM = EPT // K
TAIL = EPT - CM * K
CSPLIT = 80
KD = 128
CMD = EPT // KD
NACC = 10240
SLICE = NACC // 16
CO = 624
REM = N - 16 * CO

_mesh = plsc.VectorSubcoreMesh(core_axis_name="c", subcore_axis_name="s")


@functools.partial(
    pl.kernel,
    out_type=jax.ShapeDtypeStruct((NACC,), jnp.float32),
    mesh=_mesh,
    scratch_types=[
        pltpu.VMEM((KD,), jnp.int32),
        pltpu.VMEM((KD,), jnp.int32),
        pltpu.VMEM((KD,), jnp.int32),
        pltpu.VMEM((TAIL,), jnp.int32),
        pltpu.VMEM((KD,), jnp.float32),
        pltpu.VMEM((TAIL,), jnp.float32),
        pltpu.VMEM((SLICE,), jnp.float32),
        pltpu.VMEM_SHARED((NACC,), jnp.float32),
        pltpu.SemaphoreType.DMA,
        pltpu.SemaphoreType.DMA,
        pltpu.SemaphoreType.DMA,
        pltpu.SemaphoreType.DMA,
    ],
)
def _deg_kernel(dst_hbm_v, out_hbm, d0, d1, d2, dt, ones_buf, ones_t,
                res_buf, acc_sh, m0, m1, m2, mt):
    c = lax.axis_index("c")
    s = lax.axis_index("s")
    didx = (d0, d1, d2)
    sems = (m0, m1, m2)
    base_e = s * EPT
    for i in range(KD // 16):
        ones_buf[pl.ds(i * 16, 16)] = jnp.full((16,), 1.0, jnp.float32)
    ones_t[pl.ds(0, 16)] = jnp.full((16,), 1.0, jnp.float32)
    for i in range(SLICE // 16):
        res_buf[pl.ds(i * 16, 16)] = jnp.zeros((16,), jnp.float32)
    pltpu.sync_copy(res_buf, acc_sh.at[pl.ds(s * SLICE, SLICE)])
    plsc.subcore_barrier()

    def dst_row(j):
        return dst_hbm_v.at[pl.ds(base_e + j * KD, KD)]

    def idx_issue(j, b):
        pltpu.async_copy(dst_row(j), didx[b], sems[b])

    def idx_wait(b):
        pltpu.make_async_copy(dst_row(0), didx[b], sems[b]).wait()

    idx_issue(0, 0)
    idx_issue(1, 1)

    def body(t, carry):
        j0 = t * 3
        for b in range(3):
            j = j0 + b

            @pl.when(j + 2 < CMD)
            def _():
                idx_issue(j + 2, (b + 2) % 3)

            idx_wait(b)
            pltpu.sync_copy(ones_buf, acc_sh.at[didx[b]], add=True)
        return carry

    lax.fori_loop(0, CMD // 3, body, 0)
    pltpu.async_copy(dst_hbm_v.at[pl.ds(base_e + CMD * KD, TAIL)], dt, mt).wait()
    pltpu.sync_copy(ones_t, acc_sh.at[dt], add=True)
    plsc.subcore_barrier()
    pltpu.sync_copy(acc_sh.at[pl.ds(s * SLICE, SLICE)], res_buf)
    for i in range(SLICE // 16):
        res_buf[pl.ds(i * 16, 16)] = res_buf[pl.ds(i * 16, 16)] + 1.0

    @pl.when(c == 0)
    def _():
        pltpu.sync_copy(res_buf, out_hbm.at[pl.ds(s * SLICE, SLICE)])


def _edge_loop(g_hbm, src_row, dst_row, fix_idx, acc_sh, sidx, didx, rows,
               msi, mdi, mr, n):
    NS = 4

    def idx_issue(j, b):
        pltpu.async_copy(src_row(j), sidx[b], msi[b])
        pltpu.async_copy(dst_row(j), didx[b], mdi[b])

    def idx_wait(b):
        pltpu.make_async_copy(src_row(0), sidx[b], msi[b]).wait()
        pltpu.make_async_copy(dst_row(0), didx[b], mdi[b]).wait()

    def gather_issue(b):
        pltpu.async_copy(g_hbm.at[sidx[b]], rows[b], mr[b])

    for j in range(3):
        idx_issue(j, j)
    for j in range(2):
        idx_wait(j)
        fix_idx(j)
        gather_issue(j)

    def body(t, carry):
        j0 = t * NS
        for b in range(NS):
            j = j0 + b
            b2 = (b + 2) % NS
            b3 = (b + 3) % NS

            @pl.when(j + 3 < n)
            def _():
                idx_issue(j + 3, b3)

            @pl.when(j + 2 < n)
            def _():
                idx_wait(b2)
                fix_idx(b2)
                gather_issue(b2)

            pltpu.make_async_copy(g_hbm.at[sidx[b]], rows[b], mr[b]).wait()
            pltpu.sync_copy(rows[b], acc_sh.at[didx[b]], add=True)
        return carry

    lax.fori_loop(0, n // NS, body, 0)


_SCAT_SCRATCH = ([pltpu.VMEM((K,), jnp.int32)] * 4
                 + [pltpu.VMEM((K,), jnp.int32)] * 4
                 + [pltpu.VMEM((K, 128), jnp.float32)] * 4
                 + [pltpu.VMEM((TAIL,), jnp.int32)]
                 + [pltpu.VMEM((TAIL,), jnp.int32)]
                 + [pltpu.VMEM((TAIL, 128), jnp.float32)]
                 + [pltpu.VMEM_SHARED((NACC, 128), jnp.float32)]
                 + [pltpu.SemaphoreType.DMA] * 13)


def _tail(src_hbm, dst_hbm, g_hbm, acc_sh, st, dt, rt, mt, s, base):
    off = s * EPT + CM * K
    pltpu.async_copy(src_hbm.at[pl.ds(off, TAIL)], st, mt).wait()
    st[pl.ds(0, 16)] = st[pl.ds(0, 16)] + base
    pltpu.async_copy(dst_hbm.at[pl.ds(off, TAIL)], dt, mt).wait()
    pltpu.async_copy(g_hbm.at[st], rt, mt).wait()
    pltpu.sync_copy(rt, acc_sh.at[dt], add=True)


def _copy_out(acc_sh, out_hbm, s, base):
    pltpu.sync_copy(acc_sh.at[pl.ds(s * CO, CO)],
                    out_hbm.at[pl.ds(base + s * CO, CO)])

    @pl.when(s == 0)
    def _():
        pltpu.sync_copy(acc_sh.at[pl.ds(16 * CO, REM)],
                        out_hbm.at[pl.ds(base + 16 * CO, REM)])


@functools.partial(
    pl.kernel,
    out_type=jax.ShapeDtypeStruct((2 * N, 128), jnp.float32),
    mesh=_mesh,
    scratch_types=_SCAT_SCRATCH,
)
def _scatter_fsplit(g_hbm, src_hbm, dst_hbm, out_hbm,
                    s0, s1, s2, s3, d0, d1, d2, d3, r0, r1, r2, r3,
                    st, dt, rt, acc_sh,
                    a0, a1, a2, a3, e0, e1, e2, e3,
                    f0, f1, f2, f3, mt):
    c = lax.axis_index("c")
    s = lax.axis_index("s")
    base = c * N
    base_e = s * EPT
    sidx = (s0, s1, s2, s3)
    pltpu.sync_copy(g_hbm.at[pl.ds(base + s * CO, CO)],
                    acc_sh.at[pl.ds(s * CO, CO)])

    @pl.when(s == 0)
    def _():
        pltpu.sync_copy(g_hbm.at[pl.ds(base + 16 * CO, REM)],
                        acc_sh.at[pl.ds(16 * CO, REM)])

    plsc.subcore_barrier()

    def fix_idx(b):
        sb = sidx[b]
        for q in range(K // 16):
            sb[pl.ds(q * 16, 16)] = sb[pl.ds(q * 16, 16)] + base

    _edge_loop(g_hbm,
               lambda j: src_hbm.at[pl.ds(base_e + j * K, K)],
               lambda j: dst_hbm.at[pl.ds(base_e + j * K, K)],
               fix_idx,
               acc_sh, sidx, (d0, d1, d2, d3),
               (r0, r1, r2, r3), (a0, a1, a2, a3),
               (e0, e1, e2, e3), (f0, f1, f2, f3), CM)
    _tail(src_hbm, dst_hbm, g_hbm, acc_sh, st, dt, rt, mt, s, base)
    plsc.subcore_barrier()
    _copy_out(acc_sh, out_hbm, s, base)


@functools.partial(
    pl.kernel,
    out_type=jax.ShapeDtypeStruct((2 * N, 128), jnp.float32),
    mesh=_mesh,
    scratch_types=_SCAT_SCRATCH,
)
def _scatter_esplit(g_hbm, src_hbm, dst_hbm, out_hbm,
                    s0, s1, s2, s3, d0, d1, d2, d3, r0, r1, r2, r3,
                    st, dt, rt, acc_sh,
                    a0, a1, a2, a3, e0, e1, e2, e3,
                    f0, f1, f2, f3, mt):
    c = lax.axis_index("c")
    s = lax.axis_index("s")
    base = c * N
    j_off = c * CSPLIT
    n = CSPLIT - 4 * c
    base_e = s * EPT
    for r in range(K):
        for q in range(8):
            r0[r, pl.ds(q * 16, 16)] = jnp.zeros((16,), jnp.float32)
    for t in range(SLICE // K):
        pltpu.sync_copy(r0, acc_sh.at[pl.ds(s * SLICE + t * K, K)])
    plsc.subcore_barrier()
    _edge_loop(g_hbm,
               lambda j: src_hbm.at[pl.ds(base_e + (j_off + j) * K, K)],
               lambda j: dst_hbm.at[pl.ds(base_e + (j_off + j) * K, K)],
               lambda b: None,
               acc_sh, (s0, s1, s2, s3), (d0, d1, d2, d3),
               (r0, r1, r2, r3), (a0, a1, a2, a3),
               (e0, e1, e2, e3), (f0, f1, f2, f3), n)

    @pl.when(c == 1)
    def _():
        _tail(src_hbm, dst_hbm, g_hbm, acc_sh, st, dt, rt, mt, s, 0)

    plsc.subcore_barrier()
    _copy_out(acc_sh, out_hbm, s, base)


_RB = 1000
_RB2 = 2000


def _tc_first(x, deg, w1):
    f_in, f_out = w1.shape
    dh = f_out // 2

    def body(x_ref, deg_ref, w_ref, g_ref):
        m = jnp.dot(x_ref[...].astype(jnp.bfloat16),
                    w_ref[...].astype(jnp.bfloat16),
                    preferred_element_type=jnp.float32)
        g_ref[...] = m * lax.rsqrt(deg_ref[...])

    return pl.pallas_call(
        body,
        grid=(N // _RB2, 2),
        in_specs=[
            pl.BlockSpec((_RB2, f_in), lambda r, c: (r, 0)),
            pl.BlockSpec((_RB2, 1), lambda r, c: (r, 0)),
            pl.BlockSpec((f_in, dh), lambda r, c: (0, c)),
        ],
        out_specs=pl.BlockSpec((_RB2, dh),
                               lambda r, c: (c * (N // _RB2) + r, 0)),
        out_shape=jax.ShapeDtypeStruct((2 * N, dh), jnp.float32),
    )(x, deg, w1)


def _tc_second(s1, deg, b, w):
    f_in, f_out = w.shape

    def body(sa_ref, sb_ref, deg_ref, b_ref, w_ref, g_ref):
        dv = lax.rsqrt(deg_ref[...])
        h = jnp.concatenate([sa_ref[...], sb_ref[...]], axis=1) * dv + b_ref[...]
        h = jnp.maximum(h, 0.0)
        g_ref[...] = jnp.dot(h, w_ref[...],
                             preferred_element_type=jnp.float32) * dv

    return pl.pallas_call(
        body,
        grid=(N // _RB2,),
        in_specs=[
            pl.BlockSpec((_RB2, 128), lambda r: (r, 0)),
            pl.BlockSpec((_RB2, 128), lambda r: (N // _RB2 + r, 0)),
            pl.BlockSpec((_RB2, 1), lambda r: (r, 0)),
            pl.BlockSpec((1, f_in), lambda r: (0, 0)),
            pl.BlockSpec((f_in, f_out), lambda r: (0, 0)),
        ],
        out_specs=pl.BlockSpec((_RB2, f_out), lambda r: (r, 0)),
        out_shape=jax.ShapeDtypeStruct((N, f_out), jnp.float32),
    )(s1, s1, deg, b, w)


def _tc_third(p2, g, deg, b):
    f = g.shape[1]

    def body(pa_ref, pb_ref, g_ref, deg_ref, b_ref, o_ref):
        dv = lax.rsqrt(deg_ref[...])
        s = pa_ref[...] + pb_ref[...] + g_ref[...]
        h = jnp.maximum(s * dv + b_ref[...], 0.0)
        o_ref[...] = h * dv

    return pl.pallas_call(
        body,
        grid=(N // _RB2,),
        in_specs=[
            pl.BlockSpec((_RB2, f), lambda r: (r, 0)),
            pl.BlockSpec((_RB2, f), lambda r: (N // _RB2 + r, 0)),
            pl.BlockSpec((_RB2, f), lambda r: (r, 0)),
            pl.BlockSpec((_RB2, 1), lambda r: (r, 0)),
            pl.BlockSpec((1, f), lambda r: (0, 0)),
        ],
        out_specs=pl.BlockSpec((_RB2, f), lambda r: (r, 0)),
        out_shape=jax.ShapeDtypeStruct((N, f), jnp.float32),
    )(p2, p2, g, deg, b)


def _tc_last(p3, g, deg, w, b):
    f_in, f_out = w.shape

    def body(pa_ref, pb_ref, g_ref, deg_ref, w_ref, b_ref, o_ref):
        s = pa_ref[...] + pb_ref[...] + g_ref[...]
        m = jnp.dot(s, w_ref[...], preferred_element_type=jnp.float32)
        o_ref[...] = m * lax.rsqrt(deg_ref[...]) + b_ref[...]

    return pl.pallas_call(
        body,
        grid=(N // _RB2,),
        in_specs=[
            pl.BlockSpec((_RB2, f_in), lambda r: (r, 0)),
            pl.BlockSpec((_RB2, f_in), lambda r: (N // _RB2 + r, 0)),
            pl.BlockSpec((_RB2, f_in), lambda r: (r, 0)),
            pl.BlockSpec((_RB2, 1), lambda r: (r, 0)),
            pl.BlockSpec((f_in, f_out), lambda r: (0, 0)),
            pl.BlockSpec((1, f_out), lambda r: (0, 0)),
        ],
        out_specs=pl.BlockSpec((_RB2, f_out), lambda r: (r, 0)),
        out_shape=jax.ShapeDtypeStruct((N, f_out), jnp.float32),
    )(p3, p3, g, deg, w, b)


def kernel(x, edge_index, W1, b1, W2, b2, W3, b3):
    srcv = edge_index[0].astype(jnp.int32)
    dstv = edge_index[1].astype(jnp.int32)
    deg = _deg_kernel(dstv).reshape(NACC, 1)
    g1 = _tc_first(x, deg, W1)
    s1 = _scatter_fsplit(g1, srcv, dstv)
    g2 = _tc_second(s1, deg, b1.reshape(1, -1), W2)
    p2 = _scatter_esplit(g2, srcv, dstv)
    g3 = _tc_third(p2, g2, deg, b2.reshape(1, -1))
    p3 = _scatter_esplit(g3, srcv, dstv)
    return _tc_last(p3, g3, deg, W3, b3.reshape(1, -1))


# speedup vs baseline: 1.0574x; 1.0008x over previous
import functools

import jax
import jax.numpy as jnp
from jax import lax
from jax.experimental import pallas as pl
from jax.experimental.pallas import tpu as pltpu
from jax.experimental.pallas import tpu_sc as plsc

N = 10000
E = 160000
EPT = E // 16
K = 64
CM = EPT // K
TAIL = EPT - CM * K
CSPLIT = 80
KD = 128
CMD = EPT // KD
NACC = 10240
SLICE = NACC // 16
CO = 624
REM = N - 16 * CO

_mesh = plsc.VectorSubcoreMesh(core_axis_name="c", subcore_axis_name="s")


@functools.partial(
    pl.kernel,
    out_type=jax.ShapeDtypeStruct((NACC,), jnp.float32),
    mesh=_mesh,
    scratch_types=[
        pltpu.VMEM((KD,), jnp.int32),
        pltpu.VMEM((KD,), jnp.int32),
        pltpu.VMEM((KD,), jnp.int32),
        pltpu.VMEM((TAIL,), jnp.int32),
        pltpu.VMEM((KD,), jnp.float32),
        pltpu.VMEM((TAIL,), jnp.float32),
        pltpu.VMEM((SLICE,), jnp.float32),
        pltpu.VMEM_SHARED((NACC,), jnp.float32),
        pltpu.SemaphoreType.DMA,
        pltpu.SemaphoreType.DMA,
        pltpu.SemaphoreType.DMA,
        pltpu.SemaphoreType.DMA,
    ],
)
def _deg_kernel(dst_hbm_v, out_hbm, d0, d1, d2, dt, ones_buf, ones_t,
                res_buf, acc_sh, m0, m1, m2, mt):
    c = lax.axis_index("c")
    s = lax.axis_index("s")
    didx = (d0, d1, d2)
    sems = (m0, m1, m2)
    base_e = s * EPT
    for i in range(KD // 16):
        ones_buf[pl.ds(i * 16, 16)] = jnp.full((16,), 1.0, jnp.float32)
    ones_t[pl.ds(0, 16)] = jnp.full((16,), 1.0, jnp.float32)
    for i in range(SLICE // 16):
        res_buf[pl.ds(i * 16, 16)] = jnp.zeros((16,), jnp.float32)
    pltpu.sync_copy(res_buf, acc_sh.at[pl.ds(s * SLICE, SLICE)])
    plsc.subcore_barrier()

    def dst_row(j):
        return dst_hbm_v.at[pl.ds(base_e + j * KD, KD)]

    def idx_issue(j, b):
        pltpu.async_copy(dst_row(j), didx[b], sems[b])

    def idx_wait(b):
        pltpu.make_async_copy(dst_row(0), didx[b], sems[b]).wait()

    idx_issue(0, 0)
    idx_issue(1, 1)

    def body(t, carry):
        j0 = t * 3
        for b in range(3):
            j = j0 + b

            @pl.when(j + 2 < CMD)
            def _():
                idx_issue(j + 2, (b + 2) % 3)

            idx_wait(b)
            pltpu.sync_copy(ones_buf, acc_sh.at[didx[b]], add=True)
        return carry

    lax.fori_loop(0, CMD // 3, body, 0)
    pltpu.async_copy(dst_hbm_v.at[pl.ds(base_e + CMD * KD, TAIL)], dt, mt).wait()
    pltpu.sync_copy(ones_t, acc_sh.at[dt], add=True)
    plsc.subcore_barrier()
    pltpu.sync_copy(acc_sh.at[pl.ds(s * SLICE, SLICE)], res_buf)
    for i in range(SLICE // 16):
        res_buf[pl.ds(i * 16, 16)] = res_buf[pl.ds(i * 16, 16)] + 1.0

    @pl.when(c == 0)
    def _():
        pltpu.sync_copy(res_buf, out_hbm.at[pl.ds(s * SLICE, SLICE)])


def _edge_loop(g_hbm, src_row, dst_row, fix_idx, acc_sh, sidx, didx, rows,
               msi, mdi, mr, n):
    NS = 4

    def idx_issue(j, b):
        pltpu.async_copy(src_row(j), sidx[b], msi[b])
        pltpu.async_copy(dst_row(j), didx[b], mdi[b])

    def idx_wait(b):
        pltpu.make_async_copy(src_row(0), sidx[b], msi[b]).wait()
        pltpu.make_async_copy(dst_row(0), didx[b], mdi[b]).wait()

    def gather_issue(b):
        pltpu.async_copy(g_hbm.at[sidx[b]], rows[b], mr[b])

    for j in range(4):
        idx_issue(j, j)
    for j in range(3):
        idx_wait(j)
        fix_idx(j)
        gather_issue(j)

    def body(t, carry):
        j0 = t * NS
        for b in range(NS):
            j = j0 + b
            b3 = (b + 3) % NS

            pltpu.make_async_copy(g_hbm.at[sidx[b]], rows[b], mr[b]).wait()
            pltpu.sync_copy(rows[b], acc_sh.at[didx[b]], add=True)

            @pl.when(j + 4 < n)
            def _():
                idx_issue(j + 4, b)

            @pl.when(j + 3 < n)
            def _():
                idx_wait(b3)
                fix_idx(b3)
                gather_issue(b3)

        return carry

    lax.fori_loop(0, n // NS, body, 0)


_SCAT_SCRATCH = ([pltpu.VMEM((K,), jnp.int32)] * 4
                 + [pltpu.VMEM((K,), jnp.int32)] * 4
                 + [pltpu.VMEM((K, 128), jnp.float32)] * 4
                 + [pltpu.VMEM((TAIL,), jnp.int32)]
                 + [pltpu.VMEM((TAIL,), jnp.int32)]
                 + [pltpu.VMEM((TAIL, 128), jnp.float32)]
                 + [pltpu.VMEM_SHARED((NACC, 128), jnp.float32)]
                 + [pltpu.SemaphoreType.DMA] * 13)


def _tail(src_hbm, dst_hbm, g_hbm, acc_sh, st, dt, rt, mt, s, base):
    off = s * EPT + CM * K
    pltpu.async_copy(src_hbm.at[pl.ds(off, TAIL)], st, mt).wait()
    st[pl.ds(0, 16)] = st[pl.ds(0, 16)] + base
    pltpu.async_copy(dst_hbm.at[pl.ds(off, TAIL)], dt, mt).wait()
    pltpu.async_copy(g_hbm.at[st], rt, mt).wait()
    pltpu.sync_copy(rt, acc_sh.at[dt], add=True)


def _copy_out(acc_sh, out_hbm, s, base):
    pltpu.sync_copy(acc_sh.at[pl.ds(s * CO, CO)],
                    out_hbm.at[pl.ds(base + s * CO, CO)])

    @pl.when(s == 0)
    def _():
        pltpu.sync_copy(acc_sh.at[pl.ds(16 * CO, REM)],
                        out_hbm.at[pl.ds(base + 16 * CO, REM)])


@functools.partial(
    pl.kernel,
    out_type=jax.ShapeDtypeStruct((2 * N, 128), jnp.float32),
    mesh=_mesh,
    scratch_types=_SCAT_SCRATCH,
)
def _scatter_fsplit(g_hbm, src_hbm, dst_hbm, out_hbm,
                    s0, s1, s2, s3, d0, d1, d2, d3, r0, r1, r2, r3,
                    st, dt, rt, acc_sh,
                    a0, a1, a2, a3, e0, e1, e2, e3,
                    f0, f1, f2, f3, mt):
    c = lax.axis_index("c")
    s = lax.axis_index("s")
    base = c * N
    base_e = s * EPT
    sidx = (s0, s1, s2, s3)
    pltpu.sync_copy(g_hbm.at[pl.ds(base + s * CO, CO)],
                    acc_sh.at[pl.ds(s * CO, CO)])

    @pl.when(s == 0)
    def _():
        pltpu.sync_copy(g_hbm.at[pl.ds(base + 16 * CO, REM)],
                        acc_sh.at[pl.ds(16 * CO, REM)])

    plsc.subcore_barrier()

    def fix_idx(b):
        sb = sidx[b]
        for q in range(K // 16):
            sb[pl.ds(q * 16, 16)] = sb[pl.ds(q * 16, 16)] + base

    _edge_loop(g_hbm,
               lambda j: src_hbm.at[pl.ds(base_e + j * K, K)],
               lambda j: dst_hbm.at[pl.ds(base_e + j * K, K)],
               fix_idx,
               acc_sh, sidx, (d0, d1, d2, d3),
               (r0, r1, r2, r3), (a0, a1, a2, a3),
               (e0, e1, e2, e3), (f0, f1, f2, f3), CM)
    _tail(src_hbm, dst_hbm, g_hbm, acc_sh, st, dt, rt, mt, s, base)
    plsc.subcore_barrier()
    _copy_out(acc_sh, out_hbm, s, base)


@functools.partial(
    pl.kernel,
    out_type=jax.ShapeDtypeStruct((2 * N, 128), jnp.float32),
    mesh=_mesh,
    scratch_types=_SCAT_SCRATCH,
)
def _scatter_esplit(g_hbm, src_hbm, dst_hbm, out_hbm,
                    s0, s1, s2, s3, d0, d1, d2, d3, r0, r1, r2, r3,
                    st, dt, rt, acc_sh,
                    a0, a1, a2, a3, e0, e1, e2, e3,
                    f0, f1, f2, f3, mt):
    c = lax.axis_index("c")
    s = lax.axis_index("s")
    base = c * N
    j_off = c * CSPLIT
    n = CSPLIT - 4 * c
    base_e = s * EPT
    for r in range(K):
        for q in range(8):
            r0[r, pl.ds(q * 16, 16)] = jnp.zeros((16,), jnp.float32)
    for t in range(SLICE // K):
        pltpu.sync_copy(r0, acc_sh.at[pl.ds(s * SLICE + t * K, K)])
    plsc.subcore_barrier()
    _edge_loop(g_hbm,
               lambda j: src_hbm.at[pl.ds(base_e + (j_off + j) * K, K)],
               lambda j: dst_hbm.at[pl.ds(base_e + (j_off + j) * K, K)],
               lambda b: None,
               acc_sh, (s0, s1, s2, s3), (d0, d1, d2, d3),
               (r0, r1, r2, r3), (a0, a1, a2, a3),
               (e0, e1, e2, e3), (f0, f1, f2, f3), n)

    @pl.when(c == 1)
    def _():
        _tail(src_hbm, dst_hbm, g_hbm, acc_sh, st, dt, rt, mt, s, 0)

    plsc.subcore_barrier()
    _copy_out(acc_sh, out_hbm, s, base)


_RB = 1000
_RB2 = 2000


def _tc_first(x, deg, w1):
    f_in, f_out = w1.shape
    dh = f_out // 2

    def body(x_ref, deg_ref, w_ref, g_ref):
        m = jnp.dot(x_ref[...].astype(jnp.bfloat16),
                    w_ref[...].astype(jnp.bfloat16),
                    preferred_element_type=jnp.float32)
        g_ref[...] = m * lax.rsqrt(deg_ref[...])

    return pl.pallas_call(
        body,
        grid=(N // _RB2, 2),
        in_specs=[
            pl.BlockSpec((_RB2, f_in), lambda r, c: (r, 0)),
            pl.BlockSpec((_RB2, 1), lambda r, c: (r, 0)),
            pl.BlockSpec((f_in, dh), lambda r, c: (0, c)),
        ],
        out_specs=pl.BlockSpec((_RB2, dh),
                               lambda r, c: (c * (N // _RB2) + r, 0)),
        out_shape=jax.ShapeDtypeStruct((2 * N, dh), jnp.float32),
    )(x, deg, w1)


def _tc_second(s1, deg, b, w):
    f_in, f_out = w.shape

    def body(sa_ref, sb_ref, deg_ref, b_ref, w_ref, g_ref):
        dv = lax.rsqrt(deg_ref[...])
        h = jnp.concatenate([sa_ref[...], sb_ref[...]], axis=1) * dv + b_ref[...]
        h = jnp.maximum(h, 0.0)
        g_ref[...] = jnp.dot(h, w_ref[...],
                             preferred_element_type=jnp.float32) * dv

    return pl.pallas_call(
        body,
        grid=(N // _RB2,),
        in_specs=[
            pl.BlockSpec((_RB2, 128), lambda r: (r, 0)),
            pl.BlockSpec((_RB2, 128), lambda r: (N // _RB2 + r, 0)),
            pl.BlockSpec((_RB2, 1), lambda r: (r, 0)),
            pl.BlockSpec((1, f_in), lambda r: (0, 0)),
            pl.BlockSpec((f_in, f_out), lambda r: (0, 0)),
        ],
        out_specs=pl.BlockSpec((_RB2, f_out), lambda r: (r, 0)),
        out_shape=jax.ShapeDtypeStruct((N, f_out), jnp.float32),
    )(s1, s1, deg, b, w)


def _tc_third(p2, g, deg, b):
    f = g.shape[1]

    def body(pa_ref, pb_ref, g_ref, deg_ref, b_ref, o_ref):
        dv = lax.rsqrt(deg_ref[...])
        s = pa_ref[...] + pb_ref[...] + g_ref[...]
        h = jnp.maximum(s * dv + b_ref[...], 0.0)
        o_ref[...] = h * dv

    return pl.pallas_call(
        body,
        grid=(N // _RB2,),
        in_specs=[
            pl.BlockSpec((_RB2, f), lambda r: (r, 0)),
            pl.BlockSpec((_RB2, f), lambda r: (N // _RB2 + r, 0)),
            pl.BlockSpec((_RB2, f), lambda r: (r, 0)),
            pl.BlockSpec((_RB2, 1), lambda r: (r, 0)),
            pl.BlockSpec((1, f), lambda r: (0, 0)),
        ],
        out_specs=pl.BlockSpec((_RB2, f), lambda r: (r, 0)),
        out_shape=jax.ShapeDtypeStruct((N, f), jnp.float32),
    )(p2, p2, g, deg, b)


def _tc_last(p3, g, deg, w, b):
    f_in, f_out = w.shape

    def body(pa_ref, pb_ref, g_ref, deg_ref, w_ref, b_ref, o_ref):
        s = pa_ref[...] + pb_ref[...] + g_ref[...]
        m = jnp.dot(s, w_ref[...], preferred_element_type=jnp.float32)
        o_ref[...] = m * lax.rsqrt(deg_ref[...]) + b_ref[...]

    return pl.pallas_call(
        body,
        grid=(N // _RB2,),
        in_specs=[
            pl.BlockSpec((_RB2, f_in), lambda r: (r, 0)),
            pl.BlockSpec((_RB2, f_in), lambda r: (N // _RB2 + r, 0)),
            pl.BlockSpec((_RB2, f_in), lambda r: (r, 0)),
            pl.BlockSpec((_RB2, 1), lambda r: (r, 0)),
            pl.BlockSpec((f_in, f_out), lambda r: (0, 0)),
            pl.BlockSpec((1, f_out), lambda r: (0, 0)),
        ],
        out_specs=pl.BlockSpec((_RB2, f_out), lambda r: (r, 0)),
        out_shape=jax.ShapeDtypeStruct((N, f_out), jnp.float32),
    )(p3, p3, g, deg, w, b)


def kernel(x, edge_index, W1, b1, W2, b2, W3, b3):
    srcv = edge_index[0].astype(jnp.int32)
    dstv = edge_index[1].astype(jnp.int32)
    deg = _deg_kernel(dstv).reshape(NACC, 1)
    g1 = _tc_first(x, deg, W1)
    s1 = _scatter_fsplit(g1, srcv, dstv)
    g2 = _tc_second(s1, deg, b1.reshape(1, -1), W2)
    p2 = _scatter_esplit(g2, srcv, dstv)
    g3 = _tc_third(p2, g2, deg, b2.reshape(1, -1))
    p3 = _scatter_esplit(g3, srcv, dstv)
    return _tc_last(p3, g3, deg, W3, b3.reshape(1, -1))
